# baseline (speedup 1.0000x reference)
import jax
import jax.numpy as jnp
from jax.experimental import pallas as pl


def _ffn_kernel(x_ref, w1_ref, b1_ref, w2_ref, b2_ref, o_ref):
    h = jnp.dot(x_ref[...], w1_ref[...], preferred_element_type=jnp.float32)
    h = h + b1_ref[...]
    h = 0.5 * h * (1.0 + jax.lax.erf(h * 0.7071067811865476))
    o = jnp.dot(h.astype(jnp.bfloat16), w2_ref[...],
                preferred_element_type=jnp.float32)
    o_ref[...] = o + b2_ref[...]


def kernel(x, edge_index, W_gat, att_src, att_dst, bias_gat, ln_gamma,
           ln_beta, W1, b1, W2, b2):
    B, N, D = x.shape
    ntot = B * N
    F = W1.shape[-1]
    xf = x.reshape(ntot, D).astype(jnp.bfloat16)
    w1 = W1[0].astype(jnp.bfloat16)
    w2 = W2[0].astype(jnp.bfloat16)
    b1r = b1[0].reshape(1, F)
    b2r = b2[0].reshape(1, D)

    tn = 256
    grid = (ntot // tn,)
    out = pl.pallas_call(
        _ffn_kernel,
        grid=grid,
        in_specs=[
            pl.BlockSpec((tn, D), lambda i: (i, 0)),
            pl.BlockSpec((D, F), lambda i: (0, 0)),
            pl.BlockSpec((1, F), lambda i: (0, 0)),
            pl.BlockSpec((F, D), lambda i: (0, 0)),
            pl.BlockSpec((1, D), lambda i: (0, 0)),
        ],
        out_specs=pl.BlockSpec((tn, D), lambda i: (i, 0)),
        out_shape=jax.ShapeDtypeStruct((ntot, D), jnp.float32),
    )(xf, w1, b1r, w2, b2r)
    return out.reshape(B, N, D)


# speedup vs baseline: 1.5102x; 1.5102x over previous
import jax
import jax.numpy as jnp
from jax.experimental import pallas as pl


def _ffn_kernel(x_ref, w1_ref, b1_ref, w2_ref, b2_ref, o_ref):
    h = jnp.dot(x_ref[0], w1_ref[0], preferred_element_type=jnp.float32)
    h = h + b1_ref[0]
    h = 0.5 * h * (1.0 + jax.lax.erf(h * 0.7071067811865476))
    o = jnp.dot(h, w2_ref[0], preferred_element_type=jnp.float32)
    o_ref[0] = o + b2_ref[0]


def kernel(x, edge_index, W_gat, att_src, att_dst, bias_gat, ln_gamma,
           ln_beta, W1, b1, W2, b2):
    B, N, D = x.shape
    NE, _, F = W1.shape

    tn = 256
    grid = (B * N // tn,)
    out = pl.pallas_call(
        _ffn_kernel,
        grid=grid,
        in_specs=[
            pl.BlockSpec((1, tn, D), lambda i: (0, i, 0)),
            pl.BlockSpec((1, D, F), lambda i: (0, 0, 0)),
            pl.BlockSpec((1, 1, F), lambda i: (0, 0, 0)),
            pl.BlockSpec((1, F, D), lambda i: (0, 0, 0)),
            pl.BlockSpec((1, 1, D), lambda i: (0, 0, 0)),
        ],
        out_specs=pl.BlockSpec((1, tn, D), lambda i: (0, i, 0)),
        out_shape=jax.ShapeDtypeStruct((B, N, D), jnp.float32),
    )(x, W1, b1.reshape(NE, 1, F), W2, b2.reshape(NE, 1, D))
    return out


# speedup vs baseline: 1.5102x; 1.0000x over previous
import jax
import jax.numpy as jnp
from jax.experimental import pallas as pl
from jax.experimental.pallas import tpu as pltpu


def _ffn_kernel(x_ref, w1_ref, b1_ref, w2_ref, b2_ref, o_ref):
    h = jnp.dot(x_ref[0], w1_ref[0], preferred_element_type=jnp.float32)
    h = h + b1_ref[0]
    h = 0.5 * h * (1.0 + jax.lax.erf(h * 0.7071067811865476))
    o = jnp.dot(h, w2_ref[0], preferred_element_type=jnp.float32)
    o_ref[0] = o + b2_ref[0]


def kernel(x, edge_index, W_gat, att_src, att_dst, bias_gat, ln_gamma,
           ln_beta, W1, b1, W2, b2):
    B, N, D = x.shape
    NE, _, F = W1.shape

    tn = 256
    grid = (B * N // tn,)
    out = pl.pallas_call(
        _ffn_kernel,
        grid=grid,
        in_specs=[
            pl.BlockSpec((1, tn, D), lambda i: (0, i, 0)),
            pl.BlockSpec((1, D, F), lambda i: (0, 0, 0)),
            pl.BlockSpec((1, 1, F), lambda i: (0, 0, 0)),
            pl.BlockSpec((1, F, D), lambda i: (0, 0, 0)),
            pl.BlockSpec((1, 1, D), lambda i: (0, 0, 0)),
        ],
        out_specs=pl.BlockSpec((1, tn, D), lambda i: (0, i, 0)),
        out_shape=jax.ShapeDtypeStruct((B, N, D), jnp.float32),
        compiler_params=pltpu.CompilerParams(
            dimension_semantics=("parallel",)),
    )(x, W1, b1.reshape(NE, 1, F), W2, b2.reshape(NE, 1, D))
    return out


# speedup vs baseline: 1.5842x; 1.0490x over previous
import jax
import jax.numpy as jnp
from jax.experimental import pallas as pl
from jax.experimental.pallas import tpu as pltpu


def _ffn_kernel(x_ref, w1_ref, b1_ref, w2_ref, b2_ref, o_ref):
    h = jnp.dot(x_ref[0], w1_ref[0], preferred_element_type=jnp.float32)
    h = h + b1_ref[0]
    h = 0.5 * h * (1.0 + jax.lax.erf(h * 0.7071067811865476))
    o = jnp.dot(h, w2_ref[0], preferred_element_type=jnp.float32)
    o_ref[0] = o + b2_ref[0]


def kernel(x, edge_index, W_gat, att_src, att_dst, bias_gat, ln_gamma,
           ln_beta, W1, b1, W2, b2):
    B, N, D = x.shape
    NE, _, F = W1.shape

    tn = 512
    grid = (B * N // tn,)
    out = pl.pallas_call(
        _ffn_kernel,
        grid=grid,
        in_specs=[
            pl.BlockSpec((1, tn, D), lambda i: (0, i, 0)),
            pl.BlockSpec((1, D, F), lambda i: (0, 0, 0)),
            pl.BlockSpec((1, 1, F), lambda i: (0, 0, 0)),
            pl.BlockSpec((1, F, D), lambda i: (0, 0, 0)),
            pl.BlockSpec((1, 1, D), lambda i: (0, 0, 0)),
        ],
        out_specs=pl.BlockSpec((1, tn, D), lambda i: (0, i, 0)),
        out_shape=jax.ShapeDtypeStruct((B, N, D), jnp.float32),
        compiler_params=pltpu.CompilerParams(
            dimension_semantics=("parallel",)),
    )(x, W1, b1.reshape(NE, 1, F), W2, b2.reshape(NE, 1, D))
    return out
